# Initial kernel scaffold; baseline (speedup 1.0000x reference)
import jax
import jax.numpy as jnp
from jax.experimental import pallas as pl


def kernel(x, pos_table):
    raise NotImplementedError("write your pallas kernel here")


# speedup vs baseline: 2.2652x; 2.2652x over previous
import jax
import jax.numpy as jnp
from jax.experimental import pallas as pl


BATCH_BLOCK = 32


def _add_body(x_ref, t_ref, o_ref):
    o_ref[...] = x_ref[...] + t_ref[...][None, :, :]


def kernel(x, pos_table):
    batch, seq_len, embed = x.shape
    table = pos_table[:seq_len]
    grid = (batch // BATCH_BLOCK,)
    return pl.pallas_call(
        _add_body,
        grid=grid,
        in_specs=[
            pl.BlockSpec((BATCH_BLOCK, seq_len, embed), lambda i: (i, 0, 0)),
            pl.BlockSpec((seq_len, embed), lambda i: (0, 0)),
        ],
        out_specs=pl.BlockSpec((BATCH_BLOCK, seq_len, embed), lambda i: (i, 0, 0)),
        out_shape=jax.ShapeDtypeStruct((batch, seq_len, embed), x.dtype),
    )(x, table)
